# baseline (speedup 1.0000x reference)
import functools

import jax
import jax.numpy as jnp
from jax import lax
from jax.experimental import pallas as pl
from jax.experimental.pallas import tpu as pltpu
from jax.experimental.pallas import tpu_sc as plsc

N = 10000
NP = 10240
E = 320000
EDGES = E + N
CHUNK = 2048
NCHUNK = -(-EDGES // CHUNK)
EP = (NCHUNK + 1) * CHUNK
PADNODE = N + 100
NB = 512
NBLK = NP // NB
NW = 32
EPS_SEG = 1e-16
EPS_BN = 1e-5


def _sc_edge_kernel(F):
  FPT = F // NW
  RF = min(FPT, 4)
  NROUNDS = FPT // RF
  mesh = plsc.VectorSubcoreMesh(core_axis_name="c", subcore_axis_name="s")

  scratch = (
      [pltpu.VMEM((NP,), jnp.float32) for _ in range(RF)]
      + [pltpu.VMEM((NP,), jnp.float32) for _ in range(RF)]
      + [
          pltpu.VMEM((NP,), jnp.float32),
          pltpu.VMEM((NP,), jnp.float32),
          pltpu.VMEM((NP,), jnp.float32),
          pltpu.VMEM((2 * CHUNK,), jnp.int32),
          pltpu.VMEM((2 * CHUNK,), jnp.int32),
          pltpu.SemaphoreType.DMA,
          pltpu.SemaphoreType.DMA,
          pltpu.SemaphoreType.DMA,
          pltpu.SemaphoreType.DMA,
      ]
  )

  @functools.partial(
      pl.kernel,
      mesh=mesh,
      out_type=(
          jax.ShapeDtypeStruct((F * NP,), jnp.float32),
          jax.ShapeDtypeStruct((8 * NP,), jnp.float32),
      ),
      scratch_types=scratch,
      compiler_params=pltpu.CompilerParams(needs_layout_passes=False),
  )
  def k(xp_h, as_h, ad_h, src_h, dst_h, agg_h, s_h, *scr):
    incols = scr[:RF]
    outcols = scr[RF:2 * RF]
    (as_col, ad_col, s_col, sbuf, dbuf,
     sem_s0, sem_s1, sem_d0, sem_d1) = scr[2 * RF:]
    wid = lax.axis_index("s") * 2 + lax.axis_index("c")
    base_f = wid * FPT
    head = base_f // 64
    is_stat = jnp.equal(base_f % 64, 0)
    statf = jnp.where(is_stat, 1.0, 0.0).astype(jnp.float32)

    pltpu.sync_copy(as_h.at[pl.ds(head * NP, NP)], as_col)
    pltpu.sync_copy(ad_h.at[pl.ds(head * NP, NP)], ad_col)

    @pl.loop(0, NP // 16)
    def _(i):
      s_col[pl.ds(i * 16, 16)] = jnp.zeros((16,), jnp.float32)

    def start(g, b):
      ss = sem_s0 if b == 0 else sem_s1
      sd = sem_d0 if b == 0 else sem_d1
      pltpu.async_copy(src_h.at[pl.ds(g * CHUNK, CHUNK)],
                       sbuf.at[pl.ds(b * CHUNK, CHUNK)], ss)
      pltpu.async_copy(dst_h.at[pl.ds(g * CHUNK, CHUNK)],
                       dbuf.at[pl.ds(b * CHUNK, CHUNK)], sd)

    def wait(g, b):
      ss = sem_s0 if b == 0 else sem_s1
      sd = sem_d0 if b == 0 else sem_d1
      pltpu.make_async_copy(src_h.at[pl.ds(g * CHUNK, CHUNK)],
                            sbuf.at[pl.ds(b * CHUNK, CHUNK)], ss).wait()
      pltpu.make_async_copy(dst_h.at[pl.ds(g * CHUNK, CHUNK)],
                            dbuf.at[pl.ds(b * CHUNK, CHUNK)], sd).wait()

    for r in range(NROUNDS):
      f0 = base_f + r * RF
      for c in range(RF):
        pltpu.sync_copy(xp_h.at[pl.ds((f0 + c) * NP, NP)], incols[c])

      @pl.loop(0, NP // 16)
      def _(i):
        z = jnp.zeros((16,), jnp.float32)
        for c in range(RF):
          outcols[c][pl.ds(i * 16, 16)] = z

      def process(b):
        base = b * CHUNK

        @pl.loop(0, CHUNK // 16, unroll=8)
        def _(j):
          off = base + j * 16
          sv = sbuf[pl.ds(off, 16)]
          dv = dbuf[pl.ds(off, 16)]
          asg = plsc.load_gather(as_col, [sv])
          adg = plsc.load_gather(ad_col, [dv])
          e = asg + adg
          e = jnp.where(e >= 0.0, e, 0.2 * e)
          ex = jnp.exp(e)
          for c in range(RF):
            gv = plsc.load_gather(incols[c], [sv])
            plsc.addupdate_scatter(outcols[c], [dv], gv * ex)
          if r == 0:
            plsc.addupdate_scatter(s_col, [dv], ex * statf)

      start(0, 0)

      @pl.loop(0, NCHUNK, step=2)
      def _(g0):
        start(g0 + 1, 1)
        wait(g0, 0)
        process(0)
        start(g0 + 2, 0)
        wait(g0 + 1, 1)
        process(1)

      wait(NCHUNK, 0)

      for c in range(RF):
        pltpu.sync_copy(outcols[c], agg_h.at[pl.ds((f0 + c) * NP, NP)])

    @pl.when(is_stat)
    def _():
      pltpu.sync_copy(s_col, s_h.at[pl.ds(head * NP, NP)])

  return k


_sc_edge_256 = _sc_edge_kernel(256)
_sc_edge_64 = _sc_edge_kernel(64)


def _project(xpad, W, attS, attD, f_in, f_out):
  def body(x_ref, w_ref, s_ref, d_ref, xp_ref, as_ref, ad_ref):
    xb = x_ref[...]
    xpT = lax.dot_general(w_ref[...], xb, (((0,), (1,)), ((), ())),
                          preferred_element_type=jnp.float32)
    xp_ref[...] = xpT
    as_ref[...] = lax.dot_general(s_ref[...], xpT, (((1,), (0,)), ((), ())),
                                  preferred_element_type=jnp.float32)
    ad_ref[...] = lax.dot_general(d_ref[...], xpT, (((1,), (0,)), ((), ())),
                                  preferred_element_type=jnp.float32)

  return pl.pallas_call(
      body,
      grid=(NBLK,),
      in_specs=[
          pl.BlockSpec((NB, f_in), lambda i: (i, 0)),
          pl.BlockSpec((f_in, f_out), lambda i: (0, 0)),
          pl.BlockSpec((8, f_out), lambda i: (0, 0)),
          pl.BlockSpec((8, f_out), lambda i: (0, 0)),
      ],
      out_specs=[
          pl.BlockSpec((f_out, NB), lambda i: (0, i)),
          pl.BlockSpec((8, NB), lambda i: (0, i)),
          pl.BlockSpec((8, NB), lambda i: (0, i)),
      ],
      out_shape=[
          jax.ShapeDtypeStruct((f_out, NP), jnp.float32),
          jax.ShapeDtypeStruct((8, NP), jnp.float32),
          jax.ShapeDtypeStruct((8, NP), jnp.float32),
      ],
  )(xpad, W, attS, attD)


def _node_u(agg, sv, bias_col, f):
  nh = f // 64
  a3 = agg.reshape(nh, 64, agg.shape[-1])
  sh = sv[:nh]
  u = a3 / (sh[:, None, :] + EPS_SEG)
  return u.reshape(f, agg.shape[-1]) + bias_col


def _bn_stats(aggT, s, params, f):
  def body(agg_ref, s_ref, p_ref, st_ref):
    i = pl.program_id(0)
    u = _node_u(agg_ref[...], s_ref[...], p_ref[:, 2:3], f)
    col = i * NB + lax.broadcasted_iota(jnp.int32, (1, NB), 1)
    u = jnp.where(col < N, u, 0.0)
    s1 = jnp.sum(u, axis=1, keepdims=True)
    s2 = jnp.sum(u * u, axis=1, keepdims=True)
    blk = jnp.concatenate([s1, s2, jnp.zeros((f, 126), jnp.float32)], axis=1)

    @pl.when(i == 0)
    def _():
      st_ref[...] = jnp.zeros_like(st_ref)

    st_ref[...] += blk

  return pl.pallas_call(
      body,
      grid=(NBLK,),
      in_specs=[
          pl.BlockSpec((f, NB), lambda i: (0, i)),
          pl.BlockSpec((8, NB), lambda i: (0, i)),
          pl.BlockSpec((f, 128), lambda i: (0, 0)),
      ],
      out_specs=pl.BlockSpec((f, 128), lambda i: (0, 0)),
      out_shape=jax.ShapeDtypeStruct((f, 128), jnp.float32),
  )(aggT, s, params)


def _apply_mm(aggT, s, stats, params, Wn, attS, attD, resT, f_in, f_out,
              keep_h):
  def body(*refs):
    if resT is not None:
      (agg_ref, s_ref, st_ref, p_ref, w_ref, as_ref, ad_ref, res_ref) = refs[:8]
      outs = refs[8:]
    else:
      (agg_ref, s_ref, st_ref, p_ref, w_ref, as_ref, ad_ref) = refs[:7]
      outs = refs[7:]
    u = _node_u(agg_ref[...], s_ref[...], p_ref[:, 2:3], f_in)
    mu = st_ref[:, 0:1] / N
    var = st_ref[:, 1:2] / N - mu * mu
    hb = p_ref[:, 0:1] * (u - mu) * lax.rsqrt(var + EPS_BN) + p_ref[:, 1:2]
    hb = jnp.where(hb >= 0.0, hb, 0.01 * hb)
    if resT is not None:
      hb = hb + res_ref[...]
    xpT = lax.dot_general(w_ref[...], hb, (((0,), (0,)), ((), ())),
                          preferred_element_type=jnp.float32)
    outs[0][...] = xpT
    outs[1][...] = lax.dot_general(as_ref[...], xpT, (((1,), (0,)), ((), ())),
                                   preferred_element_type=jnp.float32)
    outs[2][...] = lax.dot_general(ad_ref[...], xpT, (((1,), (0,)), ((), ())),
                                   preferred_element_type=jnp.float32)
    if keep_h:
      outs[3][...] = hb

  in_specs = [
      pl.BlockSpec((f_in, NB), lambda i: (0, i)),
      pl.BlockSpec((8, NB), lambda i: (0, i)),
      pl.BlockSpec((f_in, 128), lambda i: (0, 0)),
      pl.BlockSpec((f_in, 128), lambda i: (0, 0)),
      pl.BlockSpec((f_in, f_out), lambda i: (0, 0)),
      pl.BlockSpec((8, f_out), lambda i: (0, 0)),
      pl.BlockSpec((8, f_out), lambda i: (0, 0)),
  ]
  args = [aggT, s, stats, params, Wn, attS, attD]
  if resT is not None:
    in_specs.append(pl.BlockSpec((f_in, NB), lambda i: (0, i)))
    args.append(resT)
  out_specs = [
      pl.BlockSpec((f_out, NB), lambda i: (0, i)),
      pl.BlockSpec((8, NB), lambda i: (0, i)),
      pl.BlockSpec((8, NB), lambda i: (0, i)),
  ]
  out_shape = [
      jax.ShapeDtypeStruct((f_out, NP), jnp.float32),
      jax.ShapeDtypeStruct((8, NP), jnp.float32),
      jax.ShapeDtypeStruct((8, NP), jnp.float32),
  ]
  if keep_h:
    out_specs.append(pl.BlockSpec((f_in, NB), lambda i: (0, i)))
    out_shape.append(jax.ShapeDtypeStruct((f_in, NP), jnp.float32))

  return pl.pallas_call(
      body,
      grid=(NBLK,),
      in_specs=in_specs,
      out_specs=out_specs,
      out_shape=out_shape,
  )(*args)


def _final(agg3, s3, params3):
  def body(agg_ref, s_ref, p_ref, o_ref):
    u = agg_ref[...] / (s_ref[0:1, :] + EPS_SEG) + p_ref[:, 2:3]
    o_ref[...] = lax.transpose(u, (1, 0))

  return pl.pallas_call(
      body,
      grid=(NBLK,),
      in_specs=[
          pl.BlockSpec((64, NB), lambda i: (0, i)),
          pl.BlockSpec((8, NB), lambda i: (0, i)),
          pl.BlockSpec((64, 128), lambda i: (0, 0)),
      ],
      out_specs=pl.BlockSpec((NB, 64), lambda i: (i, 0)),
      out_shape=jax.ShapeDtypeStruct((NP, 64), jnp.float32),
  )(agg3, s3, params3)


def _att_embed(a):
  h = a.shape[0]
  eye = jnp.eye(h, dtype=a.dtype)
  m = (eye[:, :, None] * a[None, :, :]).reshape(h, h * 64)
  return jnp.pad(m, ((0, 8 - h), (0, 0)))


def _params_cols(gamma, beta, b):
  cols = jnp.stack([gamma, beta, b], axis=1)
  return jnp.pad(cols, ((0, 0), (0, 125)))


def kernel(x, edge_index, W1, a_src1, a_dst1, b1, W2, a_src2, a_dst2, b2,
           W3, a_src3, a_dst3, b3, gamma, beta):
  loop = jnp.arange(N, dtype=jnp.int32)
  pad = jnp.full((EP - EDGES,), PADNODE, jnp.int32)
  srcp = jnp.concatenate([edge_index[0].astype(jnp.int32), loop, pad])
  dstp = jnp.concatenate([edge_index[1].astype(jnp.int32), loop, pad])

  xpad = jnp.pad(x, ((0, NP - N), (0, 0)))
  attS1, attD1 = _att_embed(a_src1), _att_embed(a_dst1)
  attS2, attD2 = _att_embed(a_src2), _att_embed(a_dst2)
  attS3, attD3 = _att_embed(a_src3), _att_embed(a_dst3)
  zeros64 = jnp.zeros((64,), jnp.float32)
  params1 = _params_cols(gamma, beta, b1)
  params2 = _params_cols(gamma, beta, b2)
  params3 = _params_cols(zeros64, zeros64, b3)

  xp1, as1, ad1 = _project(xpad, W1, attS1, attD1, 128, 256)
  agg1, s1 = _sc_edge_256(xp1.reshape(-1), as1.reshape(-1), ad1.reshape(-1),
                          srcp, dstp)
  agg1 = agg1.reshape(256, NP)
  s1 = s1.reshape(8, NP)
  stats1 = _bn_stats(agg1, s1, params1, 256)
  xp2, as2, ad2, h1 = _apply_mm(agg1, s1, stats1, params1, W2, attS2, attD2,
                                None, 256, 256, True)
  agg2, s2 = _sc_edge_256(xp2.reshape(-1), as2.reshape(-1), ad2.reshape(-1),
                          srcp, dstp)
  agg2 = agg2.reshape(256, NP)
  s2 = s2.reshape(8, NP)
  stats2 = _bn_stats(agg2, s2, params2, 256)
  xp3, as3, ad3 = _apply_mm(agg2, s2, stats2, params2, W3, attS3, attD3,
                            h1, 256, 64, False)
  agg3, s3 = _sc_edge_64(xp3.reshape(-1), as3.reshape(-1), ad3.reshape(-1),
                         srcp, dstp)
  out = _final(agg3.reshape(64, NP), s3.reshape(8, NP), params3)
  return out[:N]


# speedup vs baseline: 3.2082x; 3.2082x over previous
import functools

import jax
import jax.numpy as jnp
from jax import lax
from jax.experimental import pallas as pl
from jax.experimental.pallas import tpu as pltpu
from jax.experimental.pallas import tpu_sc as plsc

N = 10000
NP = 10240
E = 320000
EDGES = E + N
CHUNK = 2048
NCHUNK = -(-EDGES // CHUNK)
EP = (NCHUNK + 1) * CHUNK
PADNODE = N + 100
NB = 512
NBLK = NP // NB
NW = 32
EPS_SEG = 1e-16
EPS_BN = 1e-5


def _sc_edge_kernel(F):
  FPT = F // NW
  RF = min(FPT, 4)
  NROUNDS = FPT // RF
  mesh = plsc.VectorSubcoreMesh(core_axis_name="c", subcore_axis_name="s")

  scratch = (
      [pltpu.VMEM((NP,), jnp.float32) for _ in range(RF)]
      + [pltpu.VMEM((NP,), jnp.float32) for _ in range(RF)]
      + [
          pltpu.VMEM((NP,), jnp.float32),
          pltpu.VMEM((NP,), jnp.float32),
          pltpu.VMEM((NP,), jnp.float32),
          pltpu.VMEM((2 * CHUNK,), jnp.int32),
          pltpu.VMEM((2 * CHUNK,), jnp.int32),
          pltpu.SemaphoreType.DMA,
          pltpu.SemaphoreType.DMA,
          pltpu.SemaphoreType.DMA,
          pltpu.SemaphoreType.DMA,
      ]
  )

  @functools.partial(
      pl.kernel,
      mesh=mesh,
      out_type=(
          jax.ShapeDtypeStruct((F * NP,), jnp.float32),
          jax.ShapeDtypeStruct((8 * NP,), jnp.float32),
      ),
      scratch_types=scratch,
      compiler_params=pltpu.CompilerParams(needs_layout_passes=False),
  )
  def k(xp_h, as_h, ad_h, src_h, dst_h, agg_h, s_h, *scr):
    incols = scr[:RF]
    outcols = scr[RF:2 * RF]
    (as_col, ad_col, s_col, sbuf, dbuf,
     sem_s0, sem_s1, sem_d0, sem_d1) = scr[2 * RF:]
    wid = lax.axis_index("s") * 2 + lax.axis_index("c")
    base_f = wid * FPT
    head = base_f // 64
    is_stat = jnp.equal(base_f % 64, 0)
    statf = jnp.where(is_stat, 1.0, 0.0).astype(jnp.float32)

    pltpu.sync_copy(as_h.at[pl.ds(head * NP, NP)], as_col)
    pltpu.sync_copy(ad_h.at[pl.ds(head * NP, NP)], ad_col)

    @pl.loop(0, NP // 16)
    def _(i):
      s_col[pl.ds(i * 16, 16)] = jnp.zeros((16,), jnp.float32)

    def start(g, b):
      ss = sem_s0 if b == 0 else sem_s1
      sd = sem_d0 if b == 0 else sem_d1
      pltpu.async_copy(src_h.at[pl.ds(g * CHUNK, CHUNK)],
                       sbuf.at[pl.ds(b * CHUNK, CHUNK)], ss)
      pltpu.async_copy(dst_h.at[pl.ds(g * CHUNK, CHUNK)],
                       dbuf.at[pl.ds(b * CHUNK, CHUNK)], sd)

    def wait(g, b):
      ss = sem_s0 if b == 0 else sem_s1
      sd = sem_d0 if b == 0 else sem_d1
      pltpu.make_async_copy(src_h.at[pl.ds(g * CHUNK, CHUNK)],
                            sbuf.at[pl.ds(b * CHUNK, CHUNK)], ss).wait()
      pltpu.make_async_copy(dst_h.at[pl.ds(g * CHUNK, CHUNK)],
                            dbuf.at[pl.ds(b * CHUNK, CHUNK)], sd).wait()

    for r in range(NROUNDS):
      f0 = base_f + r * RF
      for c in range(RF):
        pltpu.sync_copy(xp_h.at[pl.ds((f0 + c) * NP, NP)], incols[c])

      @pl.loop(0, NP // 16)
      def _(i):
        z = jnp.zeros((16,), jnp.float32)
        for c in range(RF):
          outcols[c][pl.ds(i * 16, 16)] = z

      def process(b):
        base = b * CHUNK

        @plsc.parallel_loop(0, CHUNK // 16, unroll=4)
        def _(j):
          off = base + j * 16
          sv = sbuf[pl.ds(off, 16)]
          dv = dbuf[pl.ds(off, 16)]
          asg = plsc.load_gather(as_col, [sv])
          adg = plsc.load_gather(ad_col, [dv])
          e = asg + adg
          e = jnp.where(e >= 0.0, e, 0.2 * e)
          ex = jnp.exp(e)
          for c in range(RF):
            gv = plsc.load_gather(incols[c], [sv])
            plsc.addupdate_scatter(outcols[c], [dv], gv * ex)
          if r == 0:
            plsc.addupdate_scatter(s_col, [dv], ex * statf)

      start(0, 0)

      @pl.loop(0, NCHUNK, step=2)
      def _(g0):
        start(g0 + 1, 1)
        wait(g0, 0)
        process(0)
        start(g0 + 2, 0)
        wait(g0 + 1, 1)
        process(1)

      wait(NCHUNK, 0)

      for c in range(RF):
        pltpu.sync_copy(outcols[c], agg_h.at[pl.ds((f0 + c) * NP, NP)])

    @pl.when(is_stat)
    def _():
      pltpu.sync_copy(s_col, s_h.at[pl.ds(head * NP, NP)])

  return k


_sc_edge_256 = _sc_edge_kernel(256)
_sc_edge_64 = _sc_edge_kernel(64)


def _project(xpad, W, attS, attD, f_in, f_out):
  def body(x_ref, w_ref, s_ref, d_ref, xp_ref, as_ref, ad_ref):
    xb = x_ref[...]
    xpT = lax.dot_general(w_ref[...], xb, (((0,), (1,)), ((), ())),
                          preferred_element_type=jnp.float32)
    xp_ref[...] = xpT
    as_ref[...] = lax.dot_general(s_ref[...], xpT, (((1,), (0,)), ((), ())),
                                  preferred_element_type=jnp.float32)
    ad_ref[...] = lax.dot_general(d_ref[...], xpT, (((1,), (0,)), ((), ())),
                                  preferred_element_type=jnp.float32)

  return pl.pallas_call(
      body,
      grid=(NBLK,),
      in_specs=[
          pl.BlockSpec((NB, f_in), lambda i: (i, 0)),
          pl.BlockSpec((f_in, f_out), lambda i: (0, 0)),
          pl.BlockSpec((8, f_out), lambda i: (0, 0)),
          pl.BlockSpec((8, f_out), lambda i: (0, 0)),
      ],
      out_specs=[
          pl.BlockSpec((f_out, NB), lambda i: (0, i)),
          pl.BlockSpec((8, NB), lambda i: (0, i)),
          pl.BlockSpec((8, NB), lambda i: (0, i)),
      ],
      out_shape=[
          jax.ShapeDtypeStruct((f_out, NP), jnp.float32),
          jax.ShapeDtypeStruct((8, NP), jnp.float32),
          jax.ShapeDtypeStruct((8, NP), jnp.float32),
      ],
  )(xpad, W, attS, attD)


def _node_u(agg, sv, bias_col, f):
  nh = f // 64
  a3 = agg.reshape(nh, 64, agg.shape[-1])
  sh = sv[:nh]
  u = a3 / (sh[:, None, :] + EPS_SEG)
  return u.reshape(f, agg.shape[-1]) + bias_col


def _bn_stats(aggT, s, params, f):
  def body(agg_ref, s_ref, p_ref, st_ref):
    i = pl.program_id(0)
    u = _node_u(agg_ref[...], s_ref[...], p_ref[:, 2:3], f)
    col = i * NB + lax.broadcasted_iota(jnp.int32, (1, NB), 1)
    u = jnp.where(col < N, u, 0.0)
    s1 = jnp.sum(u, axis=1, keepdims=True)
    s2 = jnp.sum(u * u, axis=1, keepdims=True)
    blk = jnp.concatenate([s1, s2, jnp.zeros((f, 126), jnp.float32)], axis=1)

    @pl.when(i == 0)
    def _():
      st_ref[...] = jnp.zeros_like(st_ref)

    st_ref[...] += blk

  return pl.pallas_call(
      body,
      grid=(NBLK,),
      in_specs=[
          pl.BlockSpec((f, NB), lambda i: (0, i)),
          pl.BlockSpec((8, NB), lambda i: (0, i)),
          pl.BlockSpec((f, 128), lambda i: (0, 0)),
      ],
      out_specs=pl.BlockSpec((f, 128), lambda i: (0, 0)),
      out_shape=jax.ShapeDtypeStruct((f, 128), jnp.float32),
  )(aggT, s, params)


def _apply_mm(aggT, s, stats, params, Wn, attS, attD, resT, f_in, f_out,
              keep_h):
  def body(*refs):
    if resT is not None:
      (agg_ref, s_ref, st_ref, p_ref, w_ref, as_ref, ad_ref, res_ref) = refs[:8]
      outs = refs[8:]
    else:
      (agg_ref, s_ref, st_ref, p_ref, w_ref, as_ref, ad_ref) = refs[:7]
      outs = refs[7:]
    u = _node_u(agg_ref[...], s_ref[...], p_ref[:, 2:3], f_in)
    mu = st_ref[:, 0:1] / N
    var = st_ref[:, 1:2] / N - mu * mu
    hb = p_ref[:, 0:1] * (u - mu) * lax.rsqrt(var + EPS_BN) + p_ref[:, 1:2]
    hb = jnp.where(hb >= 0.0, hb, 0.01 * hb)
    if resT is not None:
      hb = hb + res_ref[...]
    xpT = lax.dot_general(w_ref[...], hb, (((0,), (0,)), ((), ())),
                          preferred_element_type=jnp.float32)
    outs[0][...] = xpT
    outs[1][...] = lax.dot_general(as_ref[...], xpT, (((1,), (0,)), ((), ())),
                                   preferred_element_type=jnp.float32)
    outs[2][...] = lax.dot_general(ad_ref[...], xpT, (((1,), (0,)), ((), ())),
                                   preferred_element_type=jnp.float32)
    if keep_h:
      outs[3][...] = hb

  in_specs = [
      pl.BlockSpec((f_in, NB), lambda i: (0, i)),
      pl.BlockSpec((8, NB), lambda i: (0, i)),
      pl.BlockSpec((f_in, 128), lambda i: (0, 0)),
      pl.BlockSpec((f_in, 128), lambda i: (0, 0)),
      pl.BlockSpec((f_in, f_out), lambda i: (0, 0)),
      pl.BlockSpec((8, f_out), lambda i: (0, 0)),
      pl.BlockSpec((8, f_out), lambda i: (0, 0)),
  ]
  args = [aggT, s, stats, params, Wn, attS, attD]
  if resT is not None:
    in_specs.append(pl.BlockSpec((f_in, NB), lambda i: (0, i)))
    args.append(resT)
  out_specs = [
      pl.BlockSpec((f_out, NB), lambda i: (0, i)),
      pl.BlockSpec((8, NB), lambda i: (0, i)),
      pl.BlockSpec((8, NB), lambda i: (0, i)),
  ]
  out_shape = [
      jax.ShapeDtypeStruct((f_out, NP), jnp.float32),
      jax.ShapeDtypeStruct((8, NP), jnp.float32),
      jax.ShapeDtypeStruct((8, NP), jnp.float32),
  ]
  if keep_h:
    out_specs.append(pl.BlockSpec((f_in, NB), lambda i: (0, i)))
    out_shape.append(jax.ShapeDtypeStruct((f_in, NP), jnp.float32))

  return pl.pallas_call(
      body,
      grid=(NBLK,),
      in_specs=in_specs,
      out_specs=out_specs,
      out_shape=out_shape,
  )(*args)


def _final(agg3, s3, params3):
  def body(agg_ref, s_ref, p_ref, o_ref):
    u = agg_ref[...] / (s_ref[0:1, :] + EPS_SEG) + p_ref[:, 2:3]
    o_ref[...] = lax.transpose(u, (1, 0))

  return pl.pallas_call(
      body,
      grid=(NBLK,),
      in_specs=[
          pl.BlockSpec((64, NB), lambda i: (0, i)),
          pl.BlockSpec((8, NB), lambda i: (0, i)),
          pl.BlockSpec((64, 128), lambda i: (0, 0)),
      ],
      out_specs=pl.BlockSpec((NB, 64), lambda i: (i, 0)),
      out_shape=jax.ShapeDtypeStruct((NP, 64), jnp.float32),
  )(agg3, s3, params3)


def _att_embed(a):
  h = a.shape[0]
  eye = jnp.eye(h, dtype=a.dtype)
  m = (eye[:, :, None] * a[None, :, :]).reshape(h, h * 64)
  return jnp.pad(m, ((0, 8 - h), (0, 0)))


def _params_cols(gamma, beta, b):
  cols = jnp.stack([gamma, beta, b], axis=1)
  return jnp.pad(cols, ((0, 0), (0, 125)))


def kernel(x, edge_index, W1, a_src1, a_dst1, b1, W2, a_src2, a_dst2, b2,
           W3, a_src3, a_dst3, b3, gamma, beta):
  loop = jnp.arange(N, dtype=jnp.int32)
  pad = jnp.full((EP - EDGES,), PADNODE, jnp.int32)
  srcp = jnp.concatenate([edge_index[0].astype(jnp.int32), loop, pad])
  dstp = jnp.concatenate([edge_index[1].astype(jnp.int32), loop, pad])

  xpad = jnp.pad(x, ((0, NP - N), (0, 0)))
  attS1, attD1 = _att_embed(a_src1), _att_embed(a_dst1)
  attS2, attD2 = _att_embed(a_src2), _att_embed(a_dst2)
  attS3, attD3 = _att_embed(a_src3), _att_embed(a_dst3)
  zeros64 = jnp.zeros((64,), jnp.float32)
  params1 = _params_cols(gamma, beta, b1)
  params2 = _params_cols(gamma, beta, b2)
  params3 = _params_cols(zeros64, zeros64, b3)

  xp1, as1, ad1 = _project(xpad, W1, attS1, attD1, 128, 256)
  agg1, s1 = _sc_edge_256(xp1.reshape(-1), as1.reshape(-1), ad1.reshape(-1),
                          srcp, dstp)
  agg1 = agg1.reshape(256, NP)
  s1 = s1.reshape(8, NP)
  stats1 = _bn_stats(agg1, s1, params1, 256)
  xp2, as2, ad2, h1 = _apply_mm(agg1, s1, stats1, params1, W2, attS2, attD2,
                                None, 256, 256, True)
  agg2, s2 = _sc_edge_256(xp2.reshape(-1), as2.reshape(-1), ad2.reshape(-1),
                          srcp, dstp)
  agg2 = agg2.reshape(256, NP)
  s2 = s2.reshape(8, NP)
  stats2 = _bn_stats(agg2, s2, params2, 256)
  xp3, as3, ad3 = _apply_mm(agg2, s2, stats2, params2, W3, attS3, attD3,
                            h1, 256, 64, False)
  agg3, s3 = _sc_edge_64(xp3.reshape(-1), as3.reshape(-1), ad3.reshape(-1),
                         srcp, dstp)
  out = _final(agg3.reshape(64, NP), s3.reshape(8, NP), params3)
  return out[:N]
